# baseline (speedup 1.0000x reference)
import functools

import jax
import jax.numpy as jnp
from jax import lax
from jax.experimental import pallas as pl
from jax.experimental.pallas import tpu as pltpu
from jax.experimental.pallas import tpu_sc as plsc

VOCAB = 4865
EMB = 8
VP = 4872
B, L_SEQ = 16384, 200

NC, NS, LANES = 2, 16, 16
NW = NC * NS
CHUNK = 2048
N_CHUNKS = B // CHUNK
L_BIG, N_BIG = 7, 8


def _fuse_body(tab_ref, w9T_ref, b9_ref, out_ref):
    t = tab_ref[...]
    out_ref[...] = jnp.dot(t, w9T_ref[...], preferred_element_type=jnp.float32) + b9_ref[...]


def _fuse_tables(table_pad, w9T, b9):
    return pl.pallas_call(
        _fuse_body,
        out_shape=jax.ShapeDtypeStruct((VP, 9), jnp.float32),
    )(table_pad, w9T, b9)


@functools.partial(
    pl.kernel,
    out_type=(
        jax.ShapeDtypeStruct((4, L_SEQ, B), jnp.float32),
        jax.ShapeDtypeStruct((5, L_SEQ, B), jnp.float32),
    ),
    mesh=plsc.VectorSubcoreMesh(core_axis_name="c", subcore_axis_name="s"),
    compiler_params=pltpu.CompilerParams(needs_layout_passes=False),
    scratch_types=[
        pltpu.VMEM((2, CHUNK), jnp.int32),
        pltpu.VMEM((VP * 9,), jnp.float32),
        pltpu.VMEM((2, 4, CHUNK), jnp.float32),
        pltpu.VMEM((2, 5, CHUNK), jnp.float32),
        pltpu.SemaphoreType.DMA,
        pltpu.SemaphoreType.DMA,
        pltpu.SemaphoreType.DMA,
        pltpu.SemaphoreType.DMA,
        pltpu.SemaphoreType.DMA,
        pltpu.SemaphoreType.DMA,
    ],
)
def _gather_kernel(xT_hbm, t9_hbm, outc_hbm, outp_hbm,
                   idxv, t9v, coorv, promv,
                   sin0, sin1, sco0, sco1, spo0, spo1):
    wid = lax.axis_index("s") * NC + lax.axis_index("c")
    is_big = wid < N_BIG
    l0 = jnp.where(is_big, L_BIG * wid, N_BIG * L_BIG + (L_BIG - 1) * (wid - N_BIG))
    nl = jnp.where(is_big, L_BIG, L_BIG - 1)
    units = nl * N_CHUNKS

    sin = [sin0, sin1]
    sco = [sco0, sco1]
    spo = [spo0, spo1]

    pltpu.sync_copy(t9_hbm, t9v)

    def l_of(u):
        return l0 + u // N_CHUNKS

    def b_of(u):
        return (u % N_CHUNKS) * CHUNK

    def start_in(u, p):
        pltpu.async_copy(
            xT_hbm.at[l_of(u), pl.ds(b_of(u), CHUNK)], idxv.at[p], sin[p])

    def gather_unit(p):
        @plsc.parallel_loop(0, CHUNK // LANES, unroll=8)
        def _(i):
            o = i * LANES
            a = idxv[p, pl.ds(o, LANES)] * 9
            g = [plsc.load_gather(t9v, [a + c]) for c in range(9)]
            for c in range(4):
                coorv[p, c, pl.ds(o, LANES)] = g[c]
            for c in range(5):
                promv[p, c, pl.ds(o, LANES)] = g[4 + c]

    def unit(u, p):
        @pl.when(u >= 2)
        def _():
            pltpu.make_async_copy(
                coorv.at[p], outc_hbm.at[:, l_of(u), pl.ds(0, CHUNK)], sco[p]).wait()
            pltpu.make_async_copy(
                promv.at[p], outp_hbm.at[:, l_of(u), pl.ds(0, CHUNK)], spo[p]).wait()
        @pl.when(u + 1 < units)
        def _():
            start_in(u + 1, 1 - p)
        pltpu.make_async_copy(
            xT_hbm.at[l_of(u), pl.ds(b_of(u), CHUNK)], idxv.at[p], sin[p]).wait()
        gather_unit(p)
        pltpu.async_copy(
            coorv.at[p], outc_hbm.at[:, l_of(u), pl.ds(b_of(u), CHUNK)], sco[p])
        pltpu.async_copy(
            promv.at[p], outp_hbm.at[:, l_of(u), pl.ds(b_of(u), CHUNK)], spo[p])

    start_in(0, 0)

    def pair(k, carry):
        unit(2 * k, 0)
        unit(2 * k + 1, 1)
        return carry

    lax.fori_loop(0, units // 2, pair, 0)

    for p in range(2):
        pltpu.make_async_copy(
            coorv.at[p], outc_hbm.at[:, 0, pl.ds(0, CHUNK)], sco[p]).wait()
        pltpu.make_async_copy(
            promv.at[p], outp_hbm.at[:, 0, pl.ds(0, CHUNK)], spo[p]).wait()


def kernel(x, table, W_coor, b_coor, W_prom, b_prom):
    table_pad = jnp.zeros((VP, EMB), jnp.float32).at[:VOCAB].set(table)
    w9T = jnp.concatenate([W_coor, W_prom], axis=0).T.astype(jnp.float32)
    b9 = jnp.concatenate([b_coor, b_prom]).reshape(1, 9).astype(jnp.float32)
    t9 = _fuse_tables(table_pad, w9T, b9)
    xT = x.T.astype(jnp.int32)
    outc_t, outp_t = _gather_kernel(xT, t9.reshape(-1))
    return jnp.transpose(outc_t, (2, 1, 0)), jnp.transpose(outp_t, (2, 1, 0))


# speedup vs baseline: 1.4034x; 1.4034x over previous
import functools

import jax
import jax.numpy as jnp
from jax import lax
from jax.experimental import pallas as pl
from jax.experimental.pallas import tpu as pltpu
from jax.experimental.pallas import tpu_sc as plsc

VOCAB = 4865
EMB = 8
VP = 4872
B, L_SEQ = 16384, 200

NC, NS, LANES = 2, 16, 16
NW = NC * NS
CHUNK = 2048
N_CHUNKS = B // CHUNK
L_BIG, N_BIG = 7, 8


def _fuse_body(tab_ref, w9T_ref, b9_ref, out_ref):
    t = tab_ref[...]
    out_ref[...] = jnp.dot(t, w9T_ref[...], preferred_element_type=jnp.float32) + b9_ref[...]


def _fuse_tables(table_pad, w9T, b9):
    return pl.pallas_call(
        _fuse_body,
        out_shape=jax.ShapeDtypeStruct((VP, 9), jnp.float32),
    )(table_pad, w9T, b9)


@functools.partial(
    pl.kernel,
    out_type=(
        jax.ShapeDtypeStruct((L_SEQ * (B // 128) * 4, 128), jnp.float32),
        jax.ShapeDtypeStruct((5, L_SEQ, B), jnp.float32),
    ),
    mesh=plsc.VectorSubcoreMesh(core_axis_name="c", subcore_axis_name="s"),
    compiler_params=pltpu.CompilerParams(needs_layout_passes=False),
    scratch_types=[
        pltpu.VMEM((2, CHUNK), jnp.int32),
        pltpu.VMEM((VP * 9,), jnp.float32),
        pltpu.VMEM((2, (CHUNK // 128) * 4, 128), jnp.float32),
        pltpu.VMEM((2, 5, CHUNK), jnp.float32),
        pltpu.SemaphoreType.DMA,
        pltpu.SemaphoreType.DMA,
        pltpu.SemaphoreType.DMA,
        pltpu.SemaphoreType.DMA,
        pltpu.SemaphoreType.DMA,
        pltpu.SemaphoreType.DMA,
    ],
)
def _gather_kernel(xT_hbm, t9_hbm, outc_hbm, outp_hbm,
                   idxv, t9v, coorv, promv,
                   sin0, sin1, sco0, sco1, spo0, spo1):
    wid = lax.axis_index("s") * NC + lax.axis_index("c")
    is_big = wid < N_BIG
    l0 = jnp.where(is_big, L_BIG * wid, N_BIG * L_BIG + (L_BIG - 1) * (wid - N_BIG))
    nl = jnp.where(is_big, L_BIG, L_BIG - 1)
    units = nl * N_CHUNKS

    sin = [sin0, sin1]
    sco = [sco0, sco1]
    spo = [spo0, spo1]

    pltpu.sync_copy(t9_hbm, t9v)

    def l_of(u):
        return l0 + u // N_CHUNKS

    def b_of(u):
        return (u % N_CHUNKS) * CHUNK

    def start_in(u, p):
        pltpu.async_copy(
            xT_hbm.at[l_of(u), pl.ds(b_of(u), CHUNK)], idxv.at[p], sin[p])

    def gather_unit(p):
        @plsc.parallel_loop(0, CHUNK // LANES, unroll=8)
        def _(i):
            o = i * LANES
            a = idxv[p, pl.ds(o, LANES)] * 9
            g = [plsc.load_gather(t9v, [a + c]) for c in range(9)]
            r0 = (i // 8) * 4
            col = (i % 8) * LANES
            for c in range(4):
                coorv[p, r0 + c, pl.ds(col, LANES)] = g[c]
            for c in range(5):
                promv[p, c, pl.ds(o, LANES)] = g[4 + c]

    def coor_rows(u):
        return pl.multiple_of((l_of(u) * (B // 128) + b_of(u) // 128) * 4, 64)

    def unit(u, p):
        @pl.when(u >= 2)
        def _():
            pltpu.make_async_copy(
                coorv.at[p], outc_hbm.at[pl.ds(0, (CHUNK // 128) * 4), :], sco[p]).wait()
            pltpu.make_async_copy(
                promv.at[p], outp_hbm.at[:, l_of(u), pl.ds(0, CHUNK)], spo[p]).wait()
        @pl.when(u + 1 < units)
        def _():
            start_in(u + 1, 1 - p)
        pltpu.make_async_copy(
            xT_hbm.at[l_of(u), pl.ds(b_of(u), CHUNK)], idxv.at[p], sin[p]).wait()
        gather_unit(p)
        pltpu.async_copy(
            coorv.at[p], outc_hbm.at[pl.ds(coor_rows(u), (CHUNK // 128) * 4), :], sco[p])
        pltpu.async_copy(
            promv.at[p], outp_hbm.at[:, l_of(u), pl.ds(b_of(u), CHUNK)], spo[p])

    start_in(0, 0)

    def pair(k, carry):
        unit(2 * k, 0)
        unit(2 * k + 1, 1)
        return carry

    lax.fori_loop(0, units // 2, pair, 0)

    for p in range(2):
        pltpu.make_async_copy(
            coorv.at[p], outc_hbm.at[pl.ds(0, (CHUNK // 128) * 4), :], sco[p]).wait()
        pltpu.make_async_copy(
            promv.at[p], outp_hbm.at[:, 0, pl.ds(0, CHUNK)], spo[p]).wait()


def kernel(x, table, W_coor, b_coor, W_prom, b_prom):
    table_pad = jnp.zeros((VP, EMB), jnp.float32).at[:VOCAB].set(table)
    w9T = jnp.concatenate([W_coor, W_prom], axis=0).T.astype(jnp.float32)
    b9 = jnp.concatenate([b_coor, b_prom]).reshape(1, 9).astype(jnp.float32)
    t9 = _fuse_tables(table_pad, w9T, b9)
    xT = x.T.astype(jnp.int32)
    outc2, outp_t = _gather_kernel(xT, t9.reshape(-1))
    outc = (
        outc2.reshape(L_SEQ, B // 128, 4, 128)
        .transpose(1, 3, 0, 2)
        .reshape(B, L_SEQ, 4)
    )
    return outc, jnp.transpose(outp_t, (2, 1, 0))


# speedup vs baseline: 1.4975x; 1.0670x over previous
import functools

import jax
import jax.numpy as jnp
from jax import lax
from jax.experimental import pallas as pl
from jax.experimental.pallas import tpu as pltpu
from jax.experimental.pallas import tpu_sc as plsc

VOCAB = 4865
EMB = 8
B, L_SEQ = 16384, 200

NC, NS, LANES = 2, 16, 16
NW = NC * NS
VPT = 384
VPX = NS * VPT
CHUNK = 2048
N_CHUNKS = B // CHUNK
L_BIG, N_BIG = 7, 8


@functools.partial(
    pl.kernel,
    out_type=(
        jax.ShapeDtypeStruct((L_SEQ * (B // 128) * 4, 128), jnp.float32),
        jax.ShapeDtypeStruct((5, L_SEQ, B), jnp.float32),
    ),
    mesh=plsc.VectorSubcoreMesh(core_axis_name="c", subcore_axis_name="s"),
    compiler_params=pltpu.CompilerParams(needs_layout_passes=False),
    scratch_types=[
        pltpu.VMEM((2, CHUNK), jnp.int32),
        pltpu.VMEM((VPX * 9,), jnp.float32),
        pltpu.VMEM((EMB, VPT), jnp.float32),
        pltpu.VMEM((2, (CHUNK // 128) * 4, 128), jnp.float32),
        pltpu.VMEM((2, 5, CHUNK), jnp.float32),
        pltpu.VMEM((96, LANES), jnp.float32),
        pltpu.VMEM_SHARED((VPX * 9,), jnp.float32),
        pltpu.SemaphoreType.DMA,
        pltpu.SemaphoreType.DMA,
        pltpu.SemaphoreType.DMA,
        pltpu.SemaphoreType.DMA,
        pltpu.SemaphoreType.DMA,
        pltpu.SemaphoreType.DMA,
    ],
)
def _moveemb_kernel(xT_hbm, tabT_hbm, wb_hbm, outc_hbm, outp_hbm,
                    idxv, t9v, tslice, coorv, promv, wbv, t9_shared,
                    sin0, sin1, sco0, sco1, spo0, spo1):
    cid = lax.axis_index("c")
    sid = lax.axis_index("s")
    wid = sid * NC + cid

    pltpu.sync_copy(wb_hbm, wbv)
    v0 = sid * VPT
    pltpu.sync_copy(tabT_hbm.at[:, pl.ds(v0, VPT)], tslice)
    iota9 = lax.iota(jnp.int32, LANES) * 9
    def splat(j):
        return wbv[j]

    for c in range(9):
        wk = [splat(c * EMB + k) for k in range(EMB)]
        bias = splat(72 + c)

        @plsc.parallel_loop(0, VPT // LANES, unroll=4)
        def _(vb):
            o = vb * LANES
            acc = tslice[0, pl.ds(o, LANES)] * wk[0]
            for k in range(1, EMB):
                acc = acc + tslice[k, pl.ds(o, LANES)] * wk[k]
            acc = acc + bias
            pos = iota9 + ((v0 + o) * 9 + c)
            plsc.store_scatter(t9v, [pos], acc)

    pltpu.sync_copy(t9v.at[pl.ds(v0 * 9, VPT * 9)],
                    t9_shared.at[pl.ds(v0 * 9, VPT * 9)])
    plsc.subcore_barrier()
    pltpu.sync_copy(t9_shared, t9v)

    is_big = wid < N_BIG
    l0 = jnp.where(is_big, L_BIG * wid, N_BIG * L_BIG + (L_BIG - 1) * (wid - N_BIG))
    nl = jnp.where(is_big, L_BIG, L_BIG - 1)
    units = nl * N_CHUNKS

    sin = [sin0, sin1]
    sco = [sco0, sco1]
    spo = [spo0, spo1]

    def l_of(u):
        return l0 + u // N_CHUNKS

    def b_of(u):
        return (u % N_CHUNKS) * CHUNK

    def start_in(u, p):
        pltpu.async_copy(
            xT_hbm.at[l_of(u), pl.ds(b_of(u), CHUNK)], idxv.at[p], sin[p])

    def gather_unit(p):
        @plsc.parallel_loop(0, CHUNK // LANES, unroll=8)
        def _(i):
            o = i * LANES
            a = idxv[p, pl.ds(o, LANES)] * 9
            g = [plsc.load_gather(t9v, [a + c]) for c in range(9)]
            r0 = (i // 8) * 4
            col = (i % 8) * LANES
            for c in range(4):
                coorv[p, r0 + c, pl.ds(col, LANES)] = g[c]
            for c in range(5):
                promv[p, c, pl.ds(o, LANES)] = g[4 + c]

    def coor_rows(u):
        return pl.multiple_of((l_of(u) * (B // 128) + b_of(u) // 128) * 4, 64)

    def unit(u, p):
        @pl.when(u >= 2)
        def _():
            pltpu.make_async_copy(
                coorv.at[p], outc_hbm.at[pl.ds(0, (CHUNK // 128) * 4), :], sco[p]).wait()
            pltpu.make_async_copy(
                promv.at[p], outp_hbm.at[:, l_of(u), pl.ds(0, CHUNK)], spo[p]).wait()
        @pl.when(u + 1 < units)
        def _():
            start_in(u + 1, 1 - p)
        pltpu.make_async_copy(
            xT_hbm.at[l_of(u), pl.ds(b_of(u), CHUNK)], idxv.at[p], sin[p]).wait()
        gather_unit(p)
        pltpu.async_copy(
            coorv.at[p], outc_hbm.at[pl.ds(coor_rows(u), (CHUNK // 128) * 4), :], sco[p])
        pltpu.async_copy(
            promv.at[p], outp_hbm.at[:, l_of(u), pl.ds(b_of(u), CHUNK)], spo[p])

    start_in(0, 0)

    def pair(k, carry):
        unit(2 * k, 0)
        unit(2 * k + 1, 1)
        return carry

    lax.fori_loop(0, units // 2, pair, 0)

    for p in range(2):
        pltpu.make_async_copy(
            coorv.at[p], outc_hbm.at[pl.ds(0, (CHUNK // 128) * 4), :], sco[p]).wait()
        pltpu.make_async_copy(
            promv.at[p], outp_hbm.at[:, 0, pl.ds(0, CHUNK)], spo[p]).wait()


def kernel(x, table, W_coor, b_coor, W_prom, b_prom):
    tabT = jnp.zeros((EMB, VPX), jnp.float32).at[:, :VOCAB].set(table.T)
    w9 = jnp.concatenate([W_coor, W_prom], axis=0).astype(jnp.float32)
    b9 = jnp.concatenate([b_coor, b_prom]).astype(jnp.float32)
    wb = jnp.zeros((96,), jnp.float32).at[:72].set(w9.reshape(-1)).at[72:81].set(b9)
    wb = jnp.tile(wb[:, None], (1, LANES))
    xT = x.T.astype(jnp.int32)
    outc2, outp_t = _moveemb_kernel(xT, tabT, wb)
    outc = (
        outc2.reshape(L_SEQ, B // 128, 4, 128)
        .transpose(1, 3, 0, 2)
        .reshape(B, L_SEQ, 4)
    )
    return outc, jnp.transpose(outp_t, (2, 1, 0))


# speedup vs baseline: 1.6169x; 1.0798x over previous
import functools

import jax
import jax.numpy as jnp
from jax import lax
from jax.experimental import pallas as pl
from jax.experimental.pallas import tpu as pltpu
from jax.experimental.pallas import tpu_sc as plsc

VOCAB = 4865
EMB = 8
B, L_SEQ = 16384, 200

NC, NS, LANES = 2, 16, 16
NW = NC * NS
VPT = 384
VPX = NS * VPT
CHUNK = 2048
N_CHUNKS = B // CHUNK
UNITS = L_SEQ * N_CHUNKS
U_PER_W = UNITS // NW


@functools.partial(
    pl.kernel,
    out_type=(
        jax.ShapeDtypeStruct((L_SEQ * (B // 128) * 4, 128), jnp.float32),
        jax.ShapeDtypeStruct((5, L_SEQ, B), jnp.float32),
    ),
    mesh=plsc.VectorSubcoreMesh(core_axis_name="c", subcore_axis_name="s"),
    compiler_params=pltpu.CompilerParams(needs_layout_passes=False),
    scratch_types=[
        pltpu.VMEM((2, CHUNK), jnp.int32),
        pltpu.VMEM((VPX * 9,), jnp.float32),
        pltpu.VMEM((EMB, VPT), jnp.float32),
        pltpu.VMEM((2, (CHUNK // 128) * 4, 128), jnp.float32),
        pltpu.VMEM((2, 5, CHUNK), jnp.float32),
        pltpu.VMEM((96, LANES), jnp.float32),
        pltpu.VMEM_SHARED((VPX * 9,), jnp.float32),
        pltpu.SemaphoreType.DMA,
        pltpu.SemaphoreType.DMA,
        pltpu.SemaphoreType.DMA,
        pltpu.SemaphoreType.DMA,
        pltpu.SemaphoreType.DMA,
        pltpu.SemaphoreType.DMA,
    ],
)
def _moveemb_kernel(xT_hbm, tabT_hbm, wb_hbm, outc_hbm, outp_hbm,
                    idxv, t9v, tslice, coorv, promv, wbv, t9_shared,
                    sin0, sin1, sco0, sco1, spo0, spo1):
    cid = lax.axis_index("c")
    sid = lax.axis_index("s")
    wid = sid * NC + cid
    u0 = wid * U_PER_W

    def l_of(u):
        return u // N_CHUNKS

    def b_of(u):
        return (u % N_CHUNKS) * CHUNK

    def start_in(u, p):
        pltpu.async_copy(
            xT_hbm.at[l_of(u), pl.ds(b_of(u), CHUNK)], idxv.at[p], sin0 if p == 0 else sin1)

    start_in(u0, 0)
    start_in(u0 + 1, 1)

    pltpu.sync_copy(wb_hbm, wbv)
    v0 = sid * VPT
    pltpu.sync_copy(tabT_hbm.at[:, pl.ds(v0, VPT)], tslice)
    iota9 = lax.iota(jnp.int32, LANES) * 9
    def splat(j):
        return wbv[j]

    for c in range(9):
        wk = [splat(c * EMB + k) for k in range(EMB)]
        bias = splat(72 + c)

        @plsc.parallel_loop(0, VPT // LANES, unroll=4)
        def _(vb):
            o = vb * LANES
            acc = tslice[0, pl.ds(o, LANES)] * wk[0]
            for k in range(1, EMB):
                acc = acc + tslice[k, pl.ds(o, LANES)] * wk[k]
            acc = acc + bias
            pos = iota9 + ((v0 + o) * 9 + c)
            plsc.store_scatter(t9v, [pos], acc)

    pltpu.sync_copy(t9v.at[pl.ds(v0 * 9, VPT * 9)],
                    t9_shared.at[pl.ds(v0 * 9, VPT * 9)])
    plsc.subcore_barrier()
    pltpu.sync_copy(t9_shared, t9v)

    sin = [sin0, sin1]
    sco = [sco0, sco1]
    spo = [spo0, spo1]

    def gather_unit(p):
        @plsc.parallel_loop(0, CHUNK // LANES, unroll=8)
        def _(i):
            o = i * LANES
            a = idxv[p, pl.ds(o, LANES)] * 9
            g = [plsc.load_gather(t9v, [a + c]) for c in range(9)]
            r0 = (i // 8) * 4
            col = (i % 8) * LANES
            for c in range(4):
                coorv[p, r0 + c, pl.ds(col, LANES)] = g[c]
            for c in range(5):
                promv[p, c, pl.ds(o, LANES)] = g[4 + c]

    def coor_rows(u):
        return pl.multiple_of((l_of(u) * (B // 128) + b_of(u) // 128) * 4, 64)

    def unit(u, p):
        @pl.when(u >= u0 + 2)
        def _():
            pltpu.make_async_copy(
                coorv.at[p], outc_hbm.at[pl.ds(0, (CHUNK // 128) * 4), :], sco[p]).wait()
            pltpu.make_async_copy(
                promv.at[p], outp_hbm.at[:, 0, pl.ds(0, CHUNK)], spo[p]).wait()
        pltpu.make_async_copy(
            xT_hbm.at[l_of(u), pl.ds(b_of(u), CHUNK)], idxv.at[p], sin[p]).wait()
        gather_unit(p)
        @pl.when(u + 2 < u0 + U_PER_W)
        def _():
            start_in(u + 2, p)
        pltpu.async_copy(
            coorv.at[p], outc_hbm.at[pl.ds(coor_rows(u), (CHUNK // 128) * 4), :], sco[p])
        pltpu.async_copy(
            promv.at[p], outp_hbm.at[:, l_of(u), pl.ds(b_of(u), CHUNK)], spo[p])

    def pair(k, carry):
        u = u0 + 2 * k
        unit(u, 0)
        unit(u + 1, 1)
        return carry

    lax.fori_loop(0, U_PER_W // 2, pair, 0)

    for p in range(2):
        pltpu.make_async_copy(
            coorv.at[p], outc_hbm.at[pl.ds(0, (CHUNK // 128) * 4), :], sco[p]).wait()
        pltpu.make_async_copy(
            promv.at[p], outp_hbm.at[:, 0, pl.ds(0, CHUNK)], spo[p]).wait()


def kernel(x, table, W_coor, b_coor, W_prom, b_prom):
    tabT = jnp.zeros((EMB, VPX), jnp.float32).at[:, :VOCAB].set(table.T)
    w9 = jnp.concatenate([W_coor, W_prom], axis=0).astype(jnp.float32)
    b9 = jnp.concatenate([b_coor, b_prom]).astype(jnp.float32)
    wb = jnp.zeros((96,), jnp.float32).at[:72].set(w9.reshape(-1)).at[72:81].set(b9)
    wb = jnp.tile(wb[:, None], (1, LANES))
    xT = x.T.astype(jnp.int32)
    outc2, outp_t = _moveemb_kernel(xT, tabT, wb)
    outc = (
        outc2.reshape(L_SEQ, B // 128, 4, 128)
        .transpose(1, 3, 0, 2)
        .reshape(B, L_SEQ, 4)
    )
    return outc, jnp.transpose(outp_t, (2, 1, 0))
